# Initial kernel scaffold; baseline (speedup 1.0000x reference)
import jax
import jax.numpy as jnp
from jax.experimental import pallas as pl


def kernel(x, W_base, A, B):
    raise NotImplementedError("write your pallas kernel here")


# speedup vs baseline: 4.6122x; 4.6122x over previous
import functools

import jax
import jax.numpy as jnp
from jax import lax
from jax.experimental import pallas as pl
from jax.experimental.pallas import tpu as pltpu
from jax.experimental.pallas import tpu_sc as plsc

_NC = 2
_NS = 16
_NW = _NC * _NS

_TC = 1024
_IDX_MINOR = 128


def _sc_embed_lora(x2d, w_base, a_t, b_t, n_tokens, embed_dim, rank):
  tokens_per_worker = n_tokens // _NW
  chunks = tokens_per_worker // _TC
  groups = _TC // _IDX_MINOR
  jblocks = embed_dim // 16

  mesh = plsc.VectorSubcoreMesh(
      core_axis_name="c", subcore_axis_name="s",
      num_cores=_NC, num_subcores=_NS)

  @functools.partial(
      pl.kernel,
      out_type=jax.ShapeDtypeStruct((n_tokens, embed_dim), jnp.float32),
      mesh=mesh,
      compiler_params=pltpu.CompilerParams(use_tc_tiling_on_sc=False),
      scratch_types=[
          pltpu.VMEM((groups, _IDX_MINOR), jnp.int32),
          pltpu.VMEM((_TC, embed_dim), jnp.float32),
          pltpu.VMEM((_TC, rank), jnp.float32),
          pltpu.VMEM((rank, embed_dim), jnp.float32),
          pltpu.SemaphoreType.DMA,
      ],
  )
  def k(x_hbm, w_hbm, at_hbm, bt_hbm, out_hbm, idx_v, base_v, a_v, bt_v, sem):
    wid = lax.axis_index("s") * _NC + lax.axis_index("c")
    pltpu.sync_copy(bt_hbm, bt_v)

    def run_chunk(c, carry):
      tok0 = pl.multiple_of(wid * tokens_per_worker + c * _TC, _TC)
      row0 = pl.multiple_of(tok0 // _IDX_MINOR, groups)
      pltpu.sync_copy(x_hbm.at[pl.ds(row0, groups)], idx_v)

      copies = []
      for g in range(groups):
        copies.append(pltpu.async_copy(
            w_hbm.at[idx_v.at[g]],
            base_v.at[pl.ds(g * _IDX_MINOR, _IDX_MINOR)], sem))
        copies.append(pltpu.async_copy(
            at_hbm.at[idx_v.at[g]],
            a_v.at[pl.ds(g * _IDX_MINOR, _IDX_MINOR)], sem))
      for cp in copies:
        cp.wait()

      for half in range(2):
        ranks = list(range(half * (rank // 2), (half + 1) * (rank // 2)))
        btv = [[bt_v[r, pl.ds(j * 16, 16)] for j in range(jblocks)]
               for r in ranks]

        def tok_body(t, tc, btv=btv, ranks=ranks):
          acc = [base_v[t, pl.ds(j * 16, 16)] for j in range(jblocks)]
          av = a_v[t, :]
          for i, r in enumerate(ranks):
            a = av[r]
            for j in range(jblocks):
              acc[j] = acc[j] + a * btv[i][j]
          for j in range(jblocks):
            base_v[t, pl.ds(j * 16, 16)] = acc[j]
          return tc

        lax.fori_loop(0, _TC, tok_body, 0, unroll=False)

      pltpu.sync_copy(base_v, out_hbm.at[pl.ds(tok0, _TC)])
      return carry

    lax.fori_loop(0, chunks, run_chunk, 0, unroll=False)

  return k(x2d, w_base, a_t, b_t)


def kernel(x, W_base, A, B):
  batch, seq = x.shape
  vocab, embed_dim = W_base.shape
  rank = A.shape[0]
  n_tokens = batch * seq

  xf = x.reshape(n_tokens).astype(jnp.int32)
  x2d = xf.reshape(n_tokens // _IDX_MINOR, _IDX_MINOR)
  a_t = A.T
  b_t = B.T

  out = _sc_embed_lora(x2d, W_base, a_t, b_t, n_tokens, embed_dim, rank)
  return out.reshape(batch, seq, embed_dim)


# speedup vs baseline: 4.7006x; 1.0192x over previous
import functools

import jax
import jax.numpy as jnp
from jax import lax
from jax.experimental import pallas as pl
from jax.experimental.pallas import tpu as pltpu
from jax.experimental.pallas import tpu_sc as plsc

_NC = 2
_NS = 16
_NW = _NC * _NS

_TC = 512
_IDX_MINOR = 128
_NBUF = 2


def _sc_embed_lora(x2d, w_base, a_t, b_t, n_tokens, embed_dim, rank):
  tokens_per_worker = n_tokens // _NW
  chunks = tokens_per_worker // _TC
  groups = _TC // _IDX_MINOR
  jblocks = embed_dim // 16

  mesh = plsc.VectorSubcoreMesh(
      core_axis_name="c", subcore_axis_name="s",
      num_cores=_NC, num_subcores=_NS)

  @functools.partial(
      pl.kernel,
      out_type=jax.ShapeDtypeStruct((n_tokens, embed_dim), jnp.float32),
      mesh=mesh,
      compiler_params=pltpu.CompilerParams(use_tc_tiling_on_sc=False),
      scratch_types=[
          pltpu.VMEM((_NBUF, groups, _IDX_MINOR), jnp.int32),
          pltpu.VMEM((_NBUF, _TC, embed_dim), jnp.float32),
          pltpu.VMEM((_NBUF, _TC, rank), jnp.float32),
          pltpu.VMEM((rank, embed_dim), jnp.float32),
          pltpu.SemaphoreType.DMA,
          pltpu.SemaphoreType.DMA,
          pltpu.SemaphoreType.DMA,
          pltpu.SemaphoreType.DMA,
      ],
  )
  def k(x_hbm, w_hbm, at_hbm, bt_hbm, out_hbm,
        idx_v, base_v, a_v, bt_v, gsem0, gsem1, osem0, osem1):
    gsems = [gsem0, gsem1]
    osems = [osem0, osem1]
    wid = lax.axis_index("s") * _NC + lax.axis_index("c")
    pltpu.sync_copy(bt_hbm, bt_v)
    worker_row0 = wid * (tokens_per_worker // _IDX_MINOR)

    def fire(c):
      b = c % _NBUF
      row0 = pl.multiple_of(worker_row0 + c * groups, groups)
      pltpu.sync_copy(x_hbm.at[pl.ds(row0, groups)], idx_v.at[b])
      cps = []
      for g in range(groups):
        cps.append(pltpu.async_copy(
            w_hbm.at[idx_v.at[b].at[g]],
            base_v.at[b].at[pl.ds(g * _IDX_MINOR, _IDX_MINOR)], gsems[b]))
        cps.append(pltpu.async_copy(
            at_hbm.at[idx_v.at[b].at[g]],
            a_v.at[b].at[pl.ds(g * _IDX_MINOR, _IDX_MINOR)], gsems[b]))
      return cps

    def compute(c):
      b = c % _NBUF
      bb = base_v.at[b]
      ab = a_v.at[b]
      for half in range(2):
        ranks = list(range(half * (rank // 2), (half + 1) * (rank // 2)))
        btv = [[bt_v[r, pl.ds(j * 16, 16)] for j in range(jblocks)]
               for r in ranks]

        def tok_body(t, tc, btv=btv, ranks=ranks, bb=bb, ab=ab):
          acc = [bb[t, pl.ds(j * 16, 16)] for j in range(jblocks)]
          av = ab[t, :]
          for i, r in enumerate(ranks):
            a = av[r]
            for j in range(jblocks):
              acc[j] = acc[j] + a * btv[i][j]
          for j in range(jblocks):
            bb[t, pl.ds(j * 16, 16)] = acc[j]
          return tc

        lax.fori_loop(0, _TC, tok_body, 0, unroll=2)

    outstanding = {}
    outcps = {}
    outstanding[0] = fire(0)
    for c in range(chunks):
      if c + 1 < chunks:
        if (c - 1) in outcps:
          outcps.pop(c - 1).wait()
        outstanding[c + 1] = fire(c + 1)
      for cp in outstanding.pop(c):
        cp.wait()
      compute(c)
      b = c % _NBUF
      tok0 = pl.multiple_of(wid * tokens_per_worker + c * _TC, _TC)
      outcps[c] = pltpu.async_copy(
          base_v.at[b], out_hbm.at[pl.ds(tok0, _TC)], osems[b])
    for cp in outcps.values():
      cp.wait()

  return k(x2d, w_base, a_t, b_t)


def kernel(x, W_base, A, B):
  batch, seq = x.shape
  vocab, embed_dim = W_base.shape
  rank = A.shape[0]
  n_tokens = batch * seq

  xf = x.reshape(n_tokens).astype(jnp.int32)
  x2d = xf.reshape(n_tokens // _IDX_MINOR, _IDX_MINOR)
  a_t = A.T
  b_t = B.T

  out = _sc_embed_lora(x2d, W_base, a_t, b_t, n_tokens, embed_dim, rank)
  return out.reshape(batch, seq, embed_dim)


# speedup vs baseline: 4.7563x; 1.0118x over previous
import functools

import jax
import jax.numpy as jnp
from jax import lax
from jax.experimental import pallas as pl
from jax.experimental.pallas import tpu as pltpu
from jax.experimental.pallas import tpu_sc as plsc

_NC = 2
_NS = 16
_NW = _NC * _NS

_TC = 256
_IDX_MINOR = 128


def _sc_embed_lora(x1d, w128, a_t, bt1d, n_tokens, embed_dim, rank):
  tokens_per_worker = n_tokens // _NW
  chunks = tokens_per_worker // _TC
  groups = _TC // _IDX_MINOR
  jblocks = embed_dim // 16
  tgroups = _TC // 16

  mesh = plsc.VectorSubcoreMesh(
      core_axis_name="c", subcore_axis_name="s",
      num_cores=_NC, num_subcores=_NS)

  @functools.partial(
      pl.kernel,
      out_type=jax.ShapeDtypeStruct((n_tokens, embed_dim), jnp.float32),
      mesh=mesh,
      compiler_params=pltpu.CompilerParams(use_tc_tiling_on_sc=False),
      scratch_types=[
          pltpu.VMEM((2, _TC), jnp.int32),
          pltpu.VMEM((2, _TC), jnp.int32),
          pltpu.VMEM((2, _TC, 2 * embed_dim), jnp.float32),
          pltpu.VMEM((2, _TC, rank), jnp.float32),
          pltpu.VMEM((2, _TC, embed_dim), jnp.float32),
          pltpu.VMEM((rank * embed_dim,), jnp.float32),
          pltpu.SemaphoreType.DMA((2,)),
          pltpu.SemaphoreType.DMA((2,)),
      ],
  )
  def k(x_hbm, w_hbm, at_hbm, bt_hbm, out_hbm,
        idx_v, idxw_v, base_v, a_v, out_v, bt_v, gsem, osem):
    wid = lax.axis_index("s") * _NC + lax.axis_index("c")
    pltpu.sync_copy(bt_hbm, bt_v)
    worker_tok0 = wid * tokens_per_worker

    def gather_parts(c, b):
      tok0 = pl.multiple_of(worker_tok0 + c * _TC, _TC)
      parts = []
      for g in range(groups):
        sl = pl.ds(g * _IDX_MINOR, _IDX_MINOR)
        parts.append((w_hbm.at[idxw_v.at[b].at[sl]],
                      base_v.at[b].at[sl], gsem.at[b]))
        parts.append((at_hbm.at[idx_v.at[b].at[sl]],
                      a_v.at[b].at[sl], gsem.at[b]))
      return tok0, parts

    def fire(c, b):
      tok0 = pl.multiple_of(worker_tok0 + c * _TC, _TC)
      pltpu.sync_copy(x_hbm.at[pl.ds(tok0, _TC)], idx_v.at[b])
      for i in range(_TC // 16):
        v = idx_v[b, pl.ds(i * 16, 16)]
        idxw_v[b, pl.ds(i * 16, 16)] = v >> 1
      _, parts = gather_parts(c, b)
      for src, dst, sem in parts:
        pltpu.async_copy(src, dst, sem)

    def wait_gathers(c, b):
      _, parts = gather_parts(c, b)
      for src, dst, sem in parts:
        pltpu.make_async_copy(src, dst, sem).wait()

    btv = [[bt_v[pl.ds(r * embed_dim + j * 16, 16)] for j in range(jblocks)]
           for r in range(rank)]

    def compute(b):
      def grp1(g, carry):
        parg = (idx_v[b, pl.ds(g * 16, 16)] & 1) * embed_dim
        for kk in range(16):
          t = g * 16 + kk
          paroff = parg[kk]
          av = a_v[b, t, :]
          acc = [base_v[b, t, pl.ds(paroff + j * 16, 16)]
                 for j in range(jblocks)]
          for r in range(rank // 2):
            a = av[r]
            for j in range(jblocks):
              acc[j] = acc[j] + a * btv[r][j]
          for j in range(jblocks):
            out_v[b, t, pl.ds(j * 16, 16)] = acc[j]
        return carry

      lax.fori_loop(0, tgroups, grp1, 0, unroll=False)

      def grp2(g, carry):
        for kk in range(16):
          t = g * 16 + kk
          av = a_v[b, t, :]
          acc = [out_v[b, t, pl.ds(j * 16, 16)] for j in range(jblocks)]
          for r in range(rank // 2, rank):
            a = av[r]
            for j in range(jblocks):
              acc[j] = acc[j] + a * btv[r][j]
          for j in range(jblocks):
            out_v[b, t, pl.ds(j * 16, 16)] = acc[j]
        return carry

      lax.fori_loop(0, tgroups, grp2, 0, unroll=False)

    def out_slice(c):
      tok0 = pl.multiple_of(worker_tok0 + c * _TC, _TC)
      return out_hbm.at[pl.ds(tok0, _TC)]

    fire(0, 0)

    def chunk_body(c, carry):
      b = lax.rem(c, 2)
      nb = 1 - b
      nc = jnp.minimum(c + 1, chunks - 1)
      fire(nc, nb)
      wait_gathers(c, b)

      @pl.when(c >= 2)
      def _():
        pltpu.make_async_copy(out_v.at[b], out_slice(c - 2), osem.at[b]).wait()

      compute(b)
      pltpu.async_copy(out_v.at[b], out_slice(c), osem.at[b])
      return carry

    lax.fori_loop(0, chunks, chunk_body, 0, unroll=False)

    wait_gathers(chunks - 1, chunks % 2)
    pltpu.make_async_copy(
        out_v.at[(chunks - 2) % 2], out_slice(chunks - 2),
        osem.at[(chunks - 2) % 2]).wait()
    pltpu.make_async_copy(
        out_v.at[(chunks - 1) % 2], out_slice(chunks - 1),
        osem.at[(chunks - 1) % 2]).wait()

  return k(x1d, w128, a_t, bt1d)


def kernel(x, W_base, A, B):
  batch, seq = x.shape
  vocab, embed_dim = W_base.shape
  rank = A.shape[0]
  n_tokens = batch * seq

  x1d = x.reshape(n_tokens).astype(jnp.int32)
  w128 = W_base.reshape(vocab // 2, 2 * embed_dim)
  a_t = A.T
  bt1d = B.T.reshape(rank * embed_dim)

  out = _sc_embed_lora(x1d, w128, a_t, bt1d, n_tokens, embed_dim, rank)
  return out.reshape(batch, seq, embed_dim)
